# Initial kernel scaffold; baseline (speedup 1.0000x reference)
import jax
import jax.numpy as jnp
from jax.experimental import pallas as pl


def kernel(queries, keys):
    raise NotImplementedError("write your pallas kernel here")


# speedup vs baseline: 3.8022x; 3.8022x over previous
import functools

import jax
import jax.numpy as jnp
from jax.experimental import pallas as pl
from jax.experimental.pallas import tpu as pltpu

Q = 1024
D = 16
K_TOTAL = 100000
BK = 2048
NSTEPS = (K_TOTAL + BK - 1) // BK


def _nn_kernel(q_ref, k_ref, score_ref, idx_ref):
    step = pl.program_id(0)

    q = q_ref[...]
    kblk = k_ref[...]

    qk = jax.lax.dot_general(
        q, kblk, (((1,), (1,)), ((), ())),
        preferred_element_type=jnp.float32)
    q_sq = jnp.sum(q * q, axis=1, keepdims=True)
    k_sq = jnp.sum(kblk * kblk, axis=1)
    dists = q_sq - 2.0 * qk + k_sq[None, :]
    dists = jnp.maximum(dists, 0.0)

    col = jax.lax.broadcasted_iota(jnp.int32, (Q, BK), 1) + step * BK
    dists = jnp.where(col < K_TOTAL, dists, jnp.inf)

    bmin = jnp.min(dists, axis=1, keepdims=True)
    bidx = jnp.min(jnp.where(dists == bmin, col, K_TOTAL),
                   axis=1, keepdims=True)

    @pl.when(step == 0)
    def _init():
        score_ref[...] = bmin
        idx_ref[...] = bidx

    @pl.when(step > 0)
    def _update():
        run = score_ref[...]
        better = bmin < run
        score_ref[...] = jnp.where(better, bmin, run)
        idx_ref[...] = jnp.where(better, bidx, idx_ref[...])

    @pl.when(step == NSTEPS - 1)
    def _finalize():
        score_ref[...] = jnp.sqrt(score_ref[...])


@functools.partial(jax.jit, static_argnames=())
def kernel(queries, keys):
    scores, nn_idx = pl.pallas_call(
        _nn_kernel,
        grid=(NSTEPS,),
        in_specs=[
            pl.BlockSpec((Q, D), lambda i: (0, 0)),
            pl.BlockSpec((BK, D), lambda i: (i, 0)),
        ],
        out_specs=[
            pl.BlockSpec((Q, 1), lambda i: (0, 0)),
            pl.BlockSpec((Q, 1), lambda i: (0, 0)),
        ],
        out_shape=[
            jax.ShapeDtypeStruct((Q, 1), jnp.float32),
            jax.ShapeDtypeStruct((Q, 1), jnp.int32),
        ],
        compiler_params=pltpu.CompilerParams(
            dimension_semantics=("arbitrary",),
        ),
    )(queries, keys)
    return scores[:, 0], nn_idx
